# baseline (speedup 1.0000x reference)
import jax
import jax.numpy as jnp
from jax.experimental import pallas as pl

NUM_TOKENS = 100000
HIDDEN_SIZE = 128
ROWS_PER_BLOCK = 25000


def _copy_block(in_ref, out_ref):
    out_ref[0] = in_ref[...]


def kernel(embedding_weight):
    grid = (NUM_TOKENS // ROWS_PER_BLOCK,)
    out = pl.pallas_call(
        _copy_block,
        grid=grid,
        in_specs=[
            pl.BlockSpec((ROWS_PER_BLOCK, HIDDEN_SIZE), lambda i: (i, 0)),
        ],
        out_specs=pl.BlockSpec((1, ROWS_PER_BLOCK, HIDDEN_SIZE), lambda i: (0, i, 0)),
        out_shape=jax.ShapeDtypeStruct((1, NUM_TOKENS, HIDDEN_SIZE), jnp.float32),
    )(embedding_weight)
    return out


# speedup vs baseline: 1.0041x; 1.0041x over previous
import jax
import jax.numpy as jnp
from jax.experimental import pallas as pl

NUM_TOKENS = 100000
HIDDEN_SIZE = 128
ROWS_PER_BLOCK = 20000


def _copy_block(in_ref, out_ref):
    out_ref[0] = in_ref[...]


def kernel(embedding_weight):
    grid = (NUM_TOKENS // ROWS_PER_BLOCK,)
    out = pl.pallas_call(
        _copy_block,
        grid=grid,
        in_specs=[
            pl.BlockSpec((ROWS_PER_BLOCK, HIDDEN_SIZE), lambda i: (i, 0)),
        ],
        out_specs=pl.BlockSpec((1, ROWS_PER_BLOCK, HIDDEN_SIZE), lambda i: (0, i, 0)),
        out_shape=jax.ShapeDtypeStruct((1, NUM_TOKENS, HIDDEN_SIZE), jnp.float32),
    )(embedding_weight)
    return out
